# baseline (speedup 1.0000x reference)
import functools

import jax
import jax.numpy as jnp
from jax import lax
from jax.experimental import pallas as pl
from jax.experimental.pallas import tpu as pltpu
from jax.experimental.pallas import tpu_sc as plsc

_NC = 2
_NS = 16
_NW = _NC * _NS
_LANES = 16


def _softplus(y):
    u = jnp.exp(-jnp.abs(y))
    z = u / (u + 2.0)
    z2 = z * z
    p = 2.0 * z * (1.0 + z2 * (1.0 / 3.0 + z2 * (1.0 / 5.0 + z2 * (1.0 / 7.0 + z2 * (1.0 / 9.0)))))
    return jnp.maximum(y, 0.0) + p


def _make_sc_kernel(V, D, B, L, C, NEG):
    NB = B // _NW
    PB = NB * C
    NC4 = D // _LANES
    NH = (C * NEG) // 2
    CIDX_ROWS = (NB * L) // 128

    mesh = plsc.VectorSubcoreMesh(
        core_axis_name="c", subcore_axis_name="s",
        num_cores=_NC, num_subcores=_NS)

    @functools.partial(
        pl.kernel,
        out_type=(
            jax.ShapeDtypeStruct((_NW, _LANES), jnp.float32),
            jax.ShapeDtypeStruct((_NW, _LANES), jnp.float32),
        ),
        mesh=mesh,
        scratch_types=[
            pltpu.VMEM((CIDX_ROWS, 128), jnp.int32),
            pltpu.VMEM((NB,), jnp.int32),
            pltpu.VMEM((NB, D), jnp.float32),
            pltpu.VMEM((NB, D), jnp.float32),
            pltpu.VMEM((NB, C), jnp.int32),
            pltpu.VMEM((PB,), jnp.int32),
            pltpu.VMEM((NB * 2, NH), jnp.int32),
            pltpu.VMEM((C, D), jnp.float32),
            pltpu.VMEM((C, D), jnp.float32),
            pltpu.VMEM((C * NEG, D), jnp.float32),
            pltpu.VMEM((C * NEG, D), jnp.float32),
            pltpu.VMEM((NEG + 1, _LANES, 17), jnp.float32),
            pltpu.VMEM((_LANES,), jnp.float32),
            pltpu.SemaphoreType.DMA,
            pltpu.SemaphoreType.DMA,
            pltpu.SemaphoreType.DMA,
        ],
        compiler_params=pltpu.CompilerParams(
            needs_layout_passes=False, use_tc_tiling_on_sc=False),
    )
    def sc_kernel(ctr_tab, ctx_tab, cw_h, lens_h, oidx_h, oflat_h, nidx_h,
                  loss_out, cnt_out,
                  cidx_v, lens_v, crows, mwe_v, oidx_v, oflat_v, nidx_v,
                  orow0, orow1, nrow0, nrow1, stage, res_v, sem0, sem1, sema):
        wid = lax.axis_index("s") * _NC + lax.axis_index("c")
        iota = lax.iota(jnp.int32, _LANES)

        pltpu.sync_copy(cw_h.at[wid], cidx_v)
        pltpu.sync_copy(lens_h.at[wid], lens_v)
        pltpu.sync_copy(oidx_h.at[wid], oidx_v)
        pltpu.sync_copy(oflat_h.at[wid], oflat_v)
        pltpu.sync_copy(nidx_h.at[wid], nidx_v)

        for j in range(CIDX_ROWS):
            pltpu.async_copy(ctr_tab.at[cidx_v.at[j]], crows, sema).wait()
            for g in range(2):
                lb32 = iota + g * _LANES
                lb = lb32 + j * 32
                lenv = lens_v[pl.ds(j * 32 + g * _LANES, _LANES)]
                recip = 1.0 / jnp.maximum(lenv.astype(jnp.float32), 1.0)
                row0 = lb32 * L

                def a_body(d, _, row0=row0, lenv=lenv, recip=recip, lb=lb):
                    dd = jnp.full((_LANES,), 0, jnp.int32) + d
                    acc = jnp.zeros((_LANES,), jnp.float32)
                    for l in range(L):
                        e = plsc.load_gather(crows, [row0 + l, dd])
                        acc = acc + jnp.where(lenv > l, e, 0.0)
                    plsc.store_scatter(mwe_v, [lb, dd], acc * recip)
                    return 0

                lax.fori_loop(0, D, a_body, 0)

        def issue(t, orow, nrow, sem):
            pltpu.async_copy(ctx_tab.at[oidx_v.at[t]], orow, sem)
            pltpu.async_copy(ctx_tab.at[nidx_v.at[2 * t]],
                             nrow.at[pl.ds(0, NH)], sem)
            pltpu.async_copy(ctx_tab.at[nidx_v.at[2 * t + 1]],
                             nrow.at[pl.ds(NH, NH)], sem)

        def drain(orow, nrow, sem):
            pltpu.make_async_copy(ctx_tab.at[pl.ds(0, C)], orow, sem).wait()
            pltpu.make_async_copy(ctx_tab.at[pl.ds(0, C * NEG)], nrow, sem).wait()

        def fold(p_hi, carry):
            lacc, cacc = carry
            scores = []
            for s in range(NEG + 1):
                srow = stage.at[s]
                acc = srow.at[0][pl.ds(0, _LANES)]
                for r in range(1, _LANES):
                    acc = acc + srow.at[r][pl.ds(0, _LANES)]
                scores.append(acc)
            ow16 = oflat_v[pl.ds(p_hi - (_LANES - 1), _LANES)]
            maskf = (ow16 != 0).astype(jnp.float32)
            ploss = _softplus(-scores[0])
            for k in range(NEG):
                ploss = ploss + _softplus(scores[k + 1])
            return (lacc + ploss * maskf, cacc + maskf)

        def compute(t, orow, nrow, carry):
            m = [mwe_v.at[t][pl.ds(c * _LANES, _LANES)] for c in range(NC4)]

            def p_body(j, carry, m=m, orow=orow, nrow=nrow):
                jm = (t * C + j) & (_LANES - 1)
                jmv = jnp.full((_LANES,), 0, jnp.int32) + jm
                orowj = orow.at[j]
                prod = m[0] * orowj[pl.ds(0, _LANES)]
                for c in range(1, NC4):
                    prod = prod + m[c] * orowj[pl.ds(c * _LANES, _LANES)]
                plsc.store_scatter(stage.at[0], [iota, jmv], prod)
                for k in range(NEG):
                    nrowk = nrow.at[j * NEG + k]
                    prod = m[0] * nrowk[pl.ds(0, _LANES)]
                    for c in range(1, NC4):
                        prod = prod + m[c] * nrowk[pl.ds(c * _LANES, _LANES)]
                    plsc.store_scatter(stage.at[k + 1], [iota, jmv], prod)

                def do_fold(carry=carry):
                    return fold(t * C + j, carry)

                def no_fold(carry=carry):
                    return carry

                return lax.cond(jm == _LANES - 1, do_fold, no_fold)

            return lax.fori_loop(0, C, p_body, carry)

        zero = jnp.zeros((_LANES,), jnp.float32)
        issue(0, orow0, nrow0, sem0)

        def chunk_body(i, carry):
            t0 = 2 * i
            issue(t0 + 1, orow1, nrow1, sem1)
            drain(orow0, nrow0, sem0)
            carry = compute(t0, orow0, nrow0, carry)

            @pl.when(t0 + 2 < NB)
            def _():
                issue(t0 + 2, orow0, nrow0, sem0)

            drain(orow1, nrow1, sem1)
            carry = compute(t0 + 1, orow1, nrow1, carry)
            return carry

        lacc, cacc = lax.fori_loop(0, NB // 2, chunk_body, (zero, zero))

        res_v[...] = lacc
        pltpu.sync_copy(res_v, loss_out.at[wid])
        res_v[...] = cacc
        pltpu.sync_copy(res_v, cnt_out.at[wid])

    return sc_kernel


def kernel(center_words, center_words_len, outside_words, negative_samples,
           center_table, context_table):
    B, L = center_words.shape
    _, C = outside_words.shape
    BC, NEG = negative_samples.shape
    V, D = center_table.shape
    NB = B // _NW
    PB = NB * C

    cw = center_words.astype(jnp.int32).reshape(_NW, (NB * L) // 128, 128)
    lens = center_words_len.astype(jnp.int32).reshape(_NW, NB)
    oidx = outside_words.astype(jnp.int32).reshape(_NW, NB, C)
    oflat = outside_words.astype(jnp.int32).reshape(_NW, PB)
    nidx = negative_samples.astype(jnp.int32).reshape(_NW, NB * 2, (C * NEG) // 2)

    f = _make_sc_kernel(V, D, B, L, C, NEG)
    loss_p, cnt_p = f(center_table, context_table, cw, lens, oidx, oflat, nidx)
    return jnp.sum(loss_p) / jnp.maximum(jnp.sum(cnt_p), 1.0)


# speedup vs baseline: 1.0013x; 1.0013x over previous
import functools

import jax
import jax.numpy as jnp
from jax import lax
from jax.experimental import pallas as pl
from jax.experimental.pallas import tpu as pltpu
from jax.experimental.pallas import tpu_sc as plsc

_NC = 2
_NS = 16
_NW = _NC * _NS
_LANES = 16


def _softplus(y):
    u = jnp.exp(-jnp.abs(y))
    z = u / (u + 2.0)
    z2 = z * z
    p = 2.0 * z * (1.0 + z2 * (1.0 / 3.0 + z2 * (1.0 / 5.0 + z2 * (1.0 / 7.0 + z2 * (1.0 / 9.0)))))
    return jnp.maximum(y, 0.0) + p


def _make_sc_kernel(V, D, B, L, C, NEG):
    NB = B // _NW
    PB = NB * C
    NC4 = D // _LANES
    NH = (C * NEG) // 2
    CIDX_ROWS = (NB * L) // 128

    mesh = plsc.VectorSubcoreMesh(
        core_axis_name="c", subcore_axis_name="s",
        num_cores=_NC, num_subcores=_NS)

    @functools.partial(
        pl.kernel,
        out_type=(
            jax.ShapeDtypeStruct((_NW, _LANES), jnp.float32),
            jax.ShapeDtypeStruct((_NW, _LANES), jnp.float32),
        ),
        mesh=mesh,
        scratch_types=[
            pltpu.VMEM((CIDX_ROWS, 128), jnp.int32),
            pltpu.VMEM((NB,), jnp.int32),
            pltpu.VMEM((NB, D), jnp.float32),
            pltpu.VMEM((NB, D), jnp.float32),
            pltpu.VMEM((NB, C), jnp.int32),
            pltpu.VMEM((PB,), jnp.int32),
            pltpu.VMEM((NB * 2, NH), jnp.int32),
            pltpu.VMEM((C, D), jnp.float32),
            pltpu.VMEM((C, D), jnp.float32),
            pltpu.VMEM((C * NEG, D), jnp.float32),
            pltpu.VMEM((C * NEG, D), jnp.float32),
            pltpu.VMEM((NEG + 1, _LANES, 17), jnp.float32),
            pltpu.VMEM((_LANES,), jnp.float32),
            pltpu.SemaphoreType.DMA,
            pltpu.SemaphoreType.DMA,
            pltpu.SemaphoreType.DMA,
        ],
        compiler_params=pltpu.CompilerParams(
            needs_layout_passes=False, use_tc_tiling_on_sc=False),
    )
    def sc_kernel(ctr_tab, ctx_tab, cw_h, lens_h, oidx_h, oflat_h, nidx_h,
                  loss_out, cnt_out,
                  cidx_v, lens_v, crows, mwe_v, oidx_v, oflat_v, nidx_v,
                  orow0, orow1, nrow0, nrow1, stage, res_v, sem0, sem1, sema):
        wid = lax.axis_index("s") * _NC + lax.axis_index("c")
        iota = lax.iota(jnp.int32, _LANES)

        pltpu.sync_copy(cw_h.at[wid], cidx_v)
        pltpu.sync_copy(lens_h.at[wid], lens_v)
        pltpu.sync_copy(oidx_h.at[wid], oidx_v)
        pltpu.sync_copy(oflat_h.at[wid], oflat_v)
        pltpu.sync_copy(nidx_h.at[wid], nidx_v)

        def issue(t, orow, nrow, sem):
            pltpu.async_copy(ctx_tab.at[oidx_v.at[t]], orow, sem)
            pltpu.async_copy(ctx_tab.at[nidx_v.at[2 * t]],
                             nrow.at[pl.ds(0, NH)], sem)
            pltpu.async_copy(ctx_tab.at[nidx_v.at[2 * t + 1]],
                             nrow.at[pl.ds(NH, NH)], sem)

        issue(0, orow0, nrow0, sem0)
        issue(1, orow1, nrow1, sem1)

        for j in range(CIDX_ROWS):
            pltpu.async_copy(ctr_tab.at[cidx_v.at[j]], crows, sema).wait()
            for g in range(2):
                lb32 = iota + g * _LANES
                lb = lb32 + j * 32
                lenv = lens_v[pl.ds(j * 32 + g * _LANES, _LANES)]
                recip = 1.0 / jnp.maximum(lenv.astype(jnp.float32), 1.0)
                row0 = lb32 * L

                def a_body(d, _, row0=row0, lenv=lenv, recip=recip, lb=lb):
                    dd = jnp.full((_LANES,), 0, jnp.int32) + d
                    acc = jnp.zeros((_LANES,), jnp.float32)
                    for l in range(L):
                        e = plsc.load_gather(crows, [row0 + l, dd])
                        acc = acc + jnp.where(lenv > l, e, 0.0)
                    plsc.store_scatter(mwe_v, [lb, dd], acc * recip)
                    return 0

                lax.fori_loop(0, D, a_body, 0)

        def drain(orow, nrow, sem):
            pltpu.make_async_copy(ctx_tab.at[pl.ds(0, C)], orow, sem).wait()
            pltpu.make_async_copy(ctx_tab.at[pl.ds(0, C * NEG)], nrow, sem).wait()

        def fold(p_hi, carry):
            lacc, cacc = carry
            scores = []
            for s in range(NEG + 1):
                srow = stage.at[s]
                acc = srow.at[0][pl.ds(0, _LANES)]
                for r in range(1, _LANES):
                    acc = acc + srow.at[r][pl.ds(0, _LANES)]
                scores.append(acc)
            ow16 = oflat_v[pl.ds(p_hi - (_LANES - 1), _LANES)]
            maskf = (ow16 != 0).astype(jnp.float32)
            ploss = _softplus(-scores[0])
            for k in range(NEG):
                ploss = ploss + _softplus(scores[k + 1])
            return (lacc + ploss * maskf, cacc + maskf)

        def compute(t, orow, nrow, carry):
            m = [mwe_v.at[t][pl.ds(c * _LANES, _LANES)] for c in range(NC4)]

            def p_body(j, carry, m=m, orow=orow, nrow=nrow):
                jm = (t * C + j) & (_LANES - 1)
                jmv = jnp.full((_LANES,), 0, jnp.int32) + jm
                orowj = orow.at[j]
                prod = m[0] * orowj[pl.ds(0, _LANES)]
                for c in range(1, NC4):
                    prod = prod + m[c] * orowj[pl.ds(c * _LANES, _LANES)]
                plsc.store_scatter(stage.at[0], [iota, jmv], prod)
                for k in range(NEG):
                    nrowk = nrow.at[j * NEG + k]
                    prod = m[0] * nrowk[pl.ds(0, _LANES)]
                    for c in range(1, NC4):
                        prod = prod + m[c] * nrowk[pl.ds(c * _LANES, _LANES)]
                    plsc.store_scatter(stage.at[k + 1], [iota, jmv], prod)

                def do_fold(carry=carry):
                    return fold(t * C + j, carry)

                def no_fold(carry=carry):
                    return carry

                return lax.cond(jm == _LANES - 1, do_fold, no_fold)

            return lax.fori_loop(0, C, p_body, carry)

        zero = jnp.zeros((_LANES,), jnp.float32)

        def chunk_body(i, carry):
            t0 = 2 * i
            drain(orow0, nrow0, sem0)
            carry = compute(t0, orow0, nrow0, carry)

            @pl.when(t0 + 2 < NB)
            def _():
                issue(t0 + 2, orow0, nrow0, sem0)

            drain(orow1, nrow1, sem1)
            carry = compute(t0 + 1, orow1, nrow1, carry)

            @pl.when(t0 + 3 < NB)
            def _():
                issue(t0 + 3, orow1, nrow1, sem1)

            return carry

        lacc, cacc = lax.fori_loop(0, NB // 2, chunk_body, (zero, zero))

        res_v[...] = lacc
        pltpu.sync_copy(res_v, loss_out.at[wid])
        res_v[...] = cacc
        pltpu.sync_copy(res_v, cnt_out.at[wid])

    return sc_kernel


def kernel(center_words, center_words_len, outside_words, negative_samples,
           center_table, context_table):
    B, L = center_words.shape
    _, C = outside_words.shape
    BC, NEG = negative_samples.shape
    V, D = center_table.shape
    NB = B // _NW
    PB = NB * C

    cw = center_words.astype(jnp.int32).reshape(_NW, (NB * L) // 128, 128)
    lens = center_words_len.astype(jnp.int32).reshape(_NW, NB)
    oidx = outside_words.astype(jnp.int32).reshape(_NW, NB, C)
    oflat = outside_words.astype(jnp.int32).reshape(_NW, PB)
    nidx = negative_samples.astype(jnp.int32).reshape(_NW, NB * 2, (C * NEG) // 2)

    f = _make_sc_kernel(V, D, B, L, C, NEG)
    loss_p, cnt_p = f(center_table, context_table, cw, lens, oidx, oflat, nidx)
    return jnp.sum(loss_p) / jnp.maximum(jnp.sum(cnt_p), 1.0)
